# baseline (speedup 1.0000x reference)
import functools

import jax
import jax.numpy as jnp
from jax import lax
from jax.experimental import pallas as pl
from jax.experimental.pallas import tpu as pltpu
from jax.experimental.pallas import tpu_sc as plsc

VOCAB = 1_000_000
DIM = 64
BATCH = 16384
NUM_CORES = 2
NUM_SUBCORES = 16
NUM_WORKERS = NUM_CORES * NUM_SUBCORES
LANE = 128
NTC_FULL = VOCAB // LANE
TAIL_LO = NTC_FULL * LANE
TAIL_W = VOCAB - TAIL_LO
NTC_BASE = NTC_FULL // NUM_WORKERS
NTC_EXTRA = NTC_FULL % NUM_WORKERS
CHUNK = 256
NCHUNK = NTC_BASE * LANE // CHUNK
NBUF = 3
NROW = 8
STAGE = 2048
SENTINEL = 0x7FFF


@functools.partial(
    pl.kernel,
    mesh=plsc.VectorSubcoreMesh(core_axis_name="c", subcore_axis_name="s"),
    out_type=jax.ShapeDtypeStruct((BATCH * DIM,), jnp.float32),
    scratch_types=[
        pltpu.VMEM((STAGE,), jnp.int32),
        pltpu.VMEM((BATCH + 64,), jnp.int32),
        pltpu.VMEM((NBUF * DIM, CHUNK), jnp.float32),
        pltpu.VMEM((DIM, LANE), jnp.float32),
        pltpu.VMEM((DIM, TAIL_W), jnp.float32),
        pltpu.VMEM((NROW * DIM,), jnp.float32),
        pltpu.SemaphoreType.DMA,
        pltpu.SemaphoreType.DMA,
    ],
    compiler_params=pltpu.CompilerParams(
        disable_bounds_checks=True, needs_layout_passes=False
    ),
)
def _sc_gather(idx_hbm, tt_hbm, out_hbm, stage_v, list_v, blk_v, bx_v,
               tail_v, rowr, sem, osem):
    wid = lax.axis_index("s") * NUM_CORES + lax.axis_index("c")
    lane16 = lax.iota(jnp.int32, 16)
    has_extra = wid < NTC_EXTRA
    is_tail_worker = wid == NUM_WORKERS - 1

    tc_lo = wid * NTC_BASE + jnp.minimum(wid, NTC_EXTRA)
    col_lo = pl.multiple_of(tc_lo * LANE, LANE)
    span = NTC_BASE * LANE
    match_hi = (col_lo + span
                + jnp.where(has_extra, LANE, 0)
                + jnp.where(is_tail_worker, TAIL_W, 0))

    def comp_stage(s, ptr):
        soff = pl.multiple_of(s * STAGE, 8)
        pltpu.sync_copy(idx_hbm.at[pl.ds(soff, STAGE)], stage_v)

        def comp_body(i, p):
            off = pl.multiple_of(i * 16, 8)
            wv = stage_v[pl.ds(off, 16)]
            m = (wv >= col_lo) & (wv < match_hi)
            pos = s * STAGE + i * 16 + lane16
            packed = (wv - col_lo) | (pos << 15)
            plsc.store_compressed(list_v.at[pl.ds(p, 16)], packed, mask=m)
            return p + jnp.max(plsc.all_reduce_population_count(m))

        return lax.fori_loop(0, STAGE // 16, comp_body, ptr)

    cnt = lax.fori_loop(0, BATCH // STAGE, comp_stage, jnp.int32(0))
    for t in range(4):
        list_v[pl.ds(cnt + t * 16, 16)] = jnp.zeros((16,), jnp.int32) + SENTINEL
    nv4 = (cnt + 63) // 64

    def fire(c):
        slot = pl.multiple_of((c % NBUF) * DIM, DIM)
        cb = pl.multiple_of(col_lo + c * CHUNK, LANE)
        pltpu.async_copy(
            tt_hbm.at[:, pl.ds(cb, CHUNK)], blk_v.at[pl.ds(slot, DIM)], sem
        )

    for b in range(NBUF):
        fire(jnp.int32(b))

    def extract_rows(kind, srcbase, lo_local, lwv, pv, m, no):
        def cond(st):
            m_, _ = st
            return jnp.max(plsc.all_reduce_population_count(m_)) > 0

        def body(st):
            m_, no_ = st
            j = jnp.max(plsc.all_reduce_ffs(m_))
            oh = lane16 == j
            lw_j = jnp.max(jnp.where(oh, lwv, 0))
            p_j = jnp.max(jnp.where(oh, pv, 0))
            colv = jnp.zeros((16,), jnp.int32) + (lw_j - lo_local)

            @pl.when(no_ >= NROW)
            def _():
                pltpu.make_async_copy(
                    rowr.at[pl.ds(0, DIM)], out_hbm.at[pl.ds(0, DIM)], osem
                ).wait()

            srow = pl.multiple_of((no_ % NROW) * DIM, DIM)
            for k in range(DIM // 16):
                rows = k * 16 + lane16
                if kind == 0:
                    g = plsc.load_gather(blk_v, [srcbase + rows, colv])
                elif kind == 1:
                    g = plsc.load_gather(bx_v, [rows, colv])
                else:
                    g = plsc.load_gather(tail_v, [rows, colv])
                rowr[pl.ds(srow + k * 16, 16)] = g
            pltpu.async_copy(
                rowr.at[pl.ds(srow, DIM)],
                out_hbm.at[pl.ds(p_j * DIM, DIM)],
                osem,
            )
            return m_ & jnp.logical_not(oh), no_ + 1

        _, no = lax.while_loop(cond, body, (m, no))
        return no

    def match_phase(kind, srcbase, lo_local, hi_local, n_out, pred=None):
        def vloop(v, no):
            lws, pvs, ms = [], [], []
            for t in range(4):
                voff = pl.multiple_of(v * 64 + t * 16, 8)
                packed = list_v[pl.ds(voff, 16)]
                lwv = packed & 0x7FFF
                pv = packed >> 15
                m = (lwv >= lo_local) & (lwv < hi_local)
                if pred is not None:
                    m = m & pred
                lws.append(lwv)
                pvs.append(pv)
                ms.append(m)
            m_any = (ms[0] | ms[1]) | (ms[2] | ms[3])

            def hit():
                no2 = no
                for t in range(4):
                    no2 = extract_rows(
                        kind, srcbase, lo_local, lws[t], pvs[t], ms[t], no2
                    )
                return no2

            return lax.cond(
                jnp.max(plsc.all_reduce_population_count(m_any)) > 0,
                hit,
                lambda: no,
            )

        return lax.fori_loop(0, nv4, vloop, n_out)

    def chunk_body(c, n_out):
        slot = pl.multiple_of((c % NBUF) * DIM, DIM)
        pltpu.make_async_copy(
            tt_hbm.at[:, pl.ds(0, CHUNK)], blk_v.at[pl.ds(slot, DIM)], sem
        ).wait()
        n_out = match_phase(0, slot, c * CHUNK, c * CHUNK + CHUNK, n_out)

        @pl.when(c + NBUF < NCHUNK)
        def _():
            fire(c + NBUF)

        return n_out

    n_out = lax.fori_loop(0, NCHUNK, chunk_body, jnp.int32(0))

    @pl.when(has_extra)
    def _():
        pltpu.sync_copy(
            tt_hbm.at[:, pl.ds(pl.multiple_of(col_lo + span, LANE), LANE)],
            bx_v,
        )

    n_out = match_phase(1, 0, span, span + LANE, n_out, pred=has_extra)

    @pl.when(is_tail_worker)
    def _():
        pltpu.sync_copy(tt_hbm.at[:, pl.ds(TAIL_LO, TAIL_W)], tail_v)

    n_out = match_phase(2, 0, span, span + TAIL_W, n_out, pred=is_tail_worker)

    def drain_body(r):
        pltpu.make_async_copy(
            rowr.at[pl.ds(0, DIM)], out_hbm.at[pl.ds(0, DIM)], osem
        ).wait()
        return r - 1

    lax.while_loop(lambda r: r > 0, drain_body, jnp.minimum(n_out, NROW))


def kernel(target_word, target_embedding):
    flat = _sc_gather(target_word.astype(jnp.int32), target_embedding.T)
    return flat.reshape(BATCH, DIM)


# speedup vs baseline: 1.0724x; 1.0724x over previous
import functools

import jax
import jax.numpy as jnp
from jax import lax
from jax.experimental import pallas as pl
from jax.experimental.pallas import tpu as pltpu
from jax.experimental.pallas import tpu_sc as plsc

VOCAB = 1_000_000
DIM = 64
BATCH = 16384
NUM_CORES = 2
NUM_SUBCORES = 16
NUM_WORKERS = NUM_CORES * NUM_SUBCORES
LANE = 128
NTC_FULL = VOCAB // LANE
TAIL_LO = NTC_FULL * LANE
TAIL_W = VOCAB - TAIL_LO
NTC_BASE = NTC_FULL // NUM_WORKERS
NTC_EXTRA = NTC_FULL % NUM_WORKERS
CHUNK = 512
NCHUNK = NTC_BASE * LANE // CHUNK
NBUF = 2
NROW = 8
STAGE = 4096
SENTINEL = 0x7FFF


@functools.partial(
    pl.kernel,
    mesh=plsc.VectorSubcoreMesh(core_axis_name="c", subcore_axis_name="s"),
    out_type=jax.ShapeDtypeStruct((BATCH * DIM,), jnp.float32),
    scratch_types=[
        pltpu.VMEM((STAGE,), jnp.int32),
        pltpu.VMEM((BATCH + 64,), jnp.int32),
        pltpu.VMEM((NBUF * DIM, CHUNK), jnp.float32),
        pltpu.VMEM((DIM, LANE), jnp.float32),
        pltpu.VMEM((DIM, TAIL_W), jnp.float32),
        pltpu.VMEM((NROW * DIM,), jnp.float32),
        pltpu.SemaphoreType.DMA,
        pltpu.SemaphoreType.DMA,
    ],
    compiler_params=pltpu.CompilerParams(
        disable_bounds_checks=True, needs_layout_passes=False
    ),
)
def _sc_gather(idx_hbm, tt_hbm, out_hbm, stage_v, list_v, blk_v, bx_v,
               tail_v, rowr, sem, osem):
    wid = lax.axis_index("s") * NUM_CORES + lax.axis_index("c")
    lane16 = lax.iota(jnp.int32, 16)
    has_extra = wid < NTC_EXTRA
    is_tail_worker = wid == NUM_WORKERS - 1

    tc_lo = wid * NTC_BASE + jnp.minimum(wid, NTC_EXTRA)
    col_lo = pl.multiple_of(tc_lo * LANE, LANE)
    span = NTC_BASE * LANE
    match_hi = (col_lo + span
                + jnp.where(has_extra, LANE, 0)
                + jnp.where(is_tail_worker, TAIL_W, 0))

    def fire(c):
        slot = pl.multiple_of((c % NBUF) * DIM, DIM)
        cb = pl.multiple_of(col_lo + c * CHUNK, LANE)
        pltpu.async_copy(
            tt_hbm.at[:, pl.ds(cb, CHUNK)], blk_v.at[pl.ds(slot, DIM)], sem
        )

    for b in range(NBUF):
        fire(jnp.int32(b))

    def comp_stage(s, ptr):
        soff = pl.multiple_of(s * STAGE, 8)
        pltpu.sync_copy(idx_hbm.at[pl.ds(soff, STAGE)], stage_v)

        def comp_body(i, p):
            off = pl.multiple_of(i * 16, 8)
            wv = stage_v[pl.ds(off, 16)]
            m = (wv >= col_lo) & (wv < match_hi)
            pos = s * STAGE + i * 16 + lane16
            packed = (wv - col_lo) | (pos << 15)
            plsc.store_compressed(list_v.at[pl.ds(p, 16)], packed, mask=m)
            return p + jnp.max(plsc.all_reduce_population_count(m))

        return lax.fori_loop(0, STAGE // 16, comp_body, ptr)

    cnt = lax.fori_loop(0, BATCH // STAGE, comp_stage, jnp.int32(0))
    for t in range(4):
        list_v[pl.ds(cnt + t * 16, 16)] = jnp.zeros((16,), jnp.int32) + SENTINEL
    nv4 = (cnt + 63) // 64

    def extract_rows(kind, srcbase, lo_local, lwv, pv, m, no):
        def cond(st):
            m_, _ = st
            return jnp.max(plsc.all_reduce_population_count(m_)) > 0

        def body(st):
            m_, no_ = st
            j = jnp.max(plsc.all_reduce_ffs(m_))
            oh = lane16 == j
            lw_j = jnp.max(jnp.where(oh, lwv, 0))
            p_j = jnp.max(jnp.where(oh, pv, 0))
            colv = jnp.zeros((16,), jnp.int32) + (lw_j - lo_local)

            @pl.when(no_ >= NROW)
            def _():
                pltpu.make_async_copy(
                    rowr.at[pl.ds(0, DIM)], out_hbm.at[pl.ds(0, DIM)], osem
                ).wait()

            srow = pl.multiple_of((no_ % NROW) * DIM, DIM)
            for k in range(DIM // 16):
                rows = k * 16 + lane16
                if kind == 0:
                    g = plsc.load_gather(blk_v, [srcbase + rows, colv])
                elif kind == 1:
                    g = plsc.load_gather(bx_v, [rows, colv])
                else:
                    g = plsc.load_gather(tail_v, [rows, colv])
                rowr[pl.ds(srow + k * 16, 16)] = g
            pltpu.async_copy(
                rowr.at[pl.ds(srow, DIM)],
                out_hbm.at[pl.ds(p_j * DIM, DIM)],
                osem,
            )
            return m_ & jnp.logical_not(oh), no_ + 1

        _, no = lax.while_loop(cond, body, (m, no))
        return no

    def match_phase(kind, srcbase, lo_local, hi_local, n_out, pred=None):
        def vloop(v, no):
            lws, pvs, ms = [], [], []
            for t in range(4):
                voff = pl.multiple_of(v * 64 + t * 16, 8)
                packed = list_v[pl.ds(voff, 16)]
                lwv = packed & 0x7FFF
                pv = packed >> 15
                m = (lwv >= lo_local) & (lwv < hi_local)
                if pred is not None:
                    m = m & pred
                lws.append(lwv)
                pvs.append(pv)
                ms.append(m)
            m_any = (ms[0] | ms[1]) | (ms[2] | ms[3])

            def hit():
                no2 = no
                for t in range(4):
                    no2 = extract_rows(
                        kind, srcbase, lo_local, lws[t], pvs[t], ms[t], no2
                    )
                return no2

            return lax.cond(
                jnp.max(plsc.all_reduce_population_count(m_any)) > 0,
                hit,
                lambda: no,
            )

        return lax.fori_loop(0, nv4, vloop, n_out)

    def chunk_body(c, n_out):
        slot = pl.multiple_of((c % NBUF) * DIM, DIM)
        pltpu.make_async_copy(
            tt_hbm.at[:, pl.ds(0, CHUNK)], blk_v.at[pl.ds(slot, DIM)], sem
        ).wait()
        n_out = match_phase(0, slot, c * CHUNK, c * CHUNK + CHUNK, n_out)

        @pl.when(c + NBUF < NCHUNK)
        def _():
            fire(c + NBUF)

        return n_out

    n_out = lax.fori_loop(0, NCHUNK, chunk_body, jnp.int32(0))

    @pl.when(has_extra)
    def _():
        pltpu.sync_copy(
            tt_hbm.at[:, pl.ds(pl.multiple_of(col_lo + span, LANE), LANE)],
            bx_v,
        )

    n_out = match_phase(1, 0, span, span + LANE, n_out, pred=has_extra)

    @pl.when(is_tail_worker)
    def _():
        pltpu.sync_copy(tt_hbm.at[:, pl.ds(TAIL_LO, TAIL_W)], tail_v)

    n_out = match_phase(2, 0, span, span + TAIL_W, n_out, pred=is_tail_worker)

    def drain_body(r):
        pltpu.make_async_copy(
            rowr.at[pl.ds(0, DIM)], out_hbm.at[pl.ds(0, DIM)], osem
        ).wait()
        return r - 1

    lax.while_loop(lambda r: r > 0, drain_body, jnp.minimum(n_out, NROW))


def kernel(target_word, target_embedding):
    flat = _sc_gather(target_word.astype(jnp.int32), target_embedding.T)
    return flat.reshape(BATCH, DIM)


# speedup vs baseline: 1.0983x; 1.0242x over previous
import functools

import jax
import jax.numpy as jnp
from jax import lax
from jax.experimental import pallas as pl
from jax.experimental.pallas import tpu as pltpu
from jax.experimental.pallas import tpu_sc as plsc

VOCAB = 1_000_000
DIM = 64
BATCH = 16384
NUM_CORES = 2
NUM_SUBCORES = 16
NUM_WORKERS = NUM_CORES * NUM_SUBCORES
LANE = 128
NTC_FULL = VOCAB // LANE
TAIL_LO = NTC_FULL * LANE
TAIL_W = VOCAB - TAIL_LO
NTC_BASE = NTC_FULL // NUM_WORKERS
NTC_EXTRA = NTC_FULL % NUM_WORKERS
CHUNK = 512
NCHUNK = NTC_BASE * LANE // CHUNK
NBUF = 2
NROW = 8
STAGE = 4096
SENTINEL = 0x7FFF


@functools.partial(
    pl.kernel,
    mesh=plsc.VectorSubcoreMesh(core_axis_name="c", subcore_axis_name="s"),
    out_type=jax.ShapeDtypeStruct((BATCH * DIM,), jnp.float32),
    scratch_types=[
        pltpu.VMEM((STAGE,), jnp.int32),
        pltpu.VMEM((BATCH + 64,), jnp.int32),
        pltpu.VMEM((NBUF * DIM, CHUNK), jnp.float32),
        pltpu.VMEM((DIM, LANE), jnp.float32),
        pltpu.VMEM((DIM, TAIL_W), jnp.float32),
        pltpu.VMEM((NROW * DIM,), jnp.float32),
        pltpu.SemaphoreType.DMA,
        pltpu.SemaphoreType.DMA,
    ],
    compiler_params=pltpu.CompilerParams(
        disable_bounds_checks=True, needs_layout_passes=False
    ),
)
def _sc_gather(idx_hbm, tt_hbm, out_hbm, stage_v, list_v, blk_v, bx_v,
               tail_v, rowr, sem, osem):
    wid = lax.axis_index("s") * NUM_CORES + lax.axis_index("c")
    lane16 = lax.iota(jnp.int32, 16)
    has_extra = wid < NTC_EXTRA
    is_tail_worker = wid == NUM_WORKERS - 1

    tc_lo = wid * NTC_BASE + jnp.minimum(wid, NTC_EXTRA)
    col_lo = pl.multiple_of(tc_lo * LANE, LANE)
    span = NTC_BASE * LANE
    match_hi = (col_lo + span
                + jnp.where(has_extra, LANE, 0)
                + jnp.where(is_tail_worker, TAIL_W, 0))

    def fire(c):
        slot = pl.multiple_of((c % NBUF) * DIM, DIM)
        cb = pl.multiple_of(col_lo + c * CHUNK, LANE)
        pltpu.async_copy(
            tt_hbm.at[:, pl.ds(cb, CHUNK)], blk_v.at[pl.ds(slot, DIM)], sem
        )

    for b in range(NBUF):
        fire(jnp.int32(b))

    def comp_stage(s, ptr):
        soff = pl.multiple_of(s * STAGE, 8)
        pltpu.sync_copy(idx_hbm.at[pl.ds(soff, STAGE)], stage_v)

        def comp_body(i, p):
            packs, masks, cnts = [], [], []
            for t in range(4):
                off = pl.multiple_of(i * 64 + t * 16, 8)
                wv = stage_v[pl.ds(off, 16)]
                m = (wv >= col_lo) & (wv < match_hi)
                pos = s * STAGE + i * 64 + t * 16 + lane16
                packs.append((wv - col_lo) | (pos << 15))
                masks.append(m)
                cnts.append(jnp.max(plsc.all_reduce_population_count(m)))
            for t in range(4):
                plsc.store_compressed(
                    list_v.at[pl.ds(p, 16)], packs[t], mask=masks[t]
                )
                p = p + cnts[t]
            return p

        return lax.fori_loop(0, STAGE // 64, comp_body, ptr)

    cnt = lax.fori_loop(0, BATCH // STAGE, comp_stage, jnp.int32(0))
    for t in range(4):
        list_v[pl.ds(cnt + t * 16, 16)] = jnp.zeros((16,), jnp.int32) + SENTINEL
    nv4 = (cnt + 63) // 64

    def extract_rows(kind, srcbase, lo_local, lwv, pv, m, no):
        def cond(st):
            m_, _ = st
            return jnp.max(plsc.all_reduce_population_count(m_)) > 0

        def body(st):
            m_, no_ = st
            j = jnp.max(plsc.all_reduce_ffs(m_))
            oh = lane16 == j
            lw_j = jnp.max(jnp.where(oh, lwv, 0))
            p_j = jnp.max(jnp.where(oh, pv, 0))
            colv = jnp.zeros((16,), jnp.int32) + (lw_j - lo_local)

            @pl.when(no_ >= NROW)
            def _():
                pltpu.make_async_copy(
                    rowr.at[pl.ds(0, DIM)], out_hbm.at[pl.ds(0, DIM)], osem
                ).wait()

            srow = pl.multiple_of((no_ % NROW) * DIM, DIM)
            for k in range(DIM // 16):
                rows = k * 16 + lane16
                if kind == 0:
                    g = plsc.load_gather(blk_v, [srcbase + rows, colv])
                elif kind == 1:
                    g = plsc.load_gather(bx_v, [rows, colv])
                else:
                    g = plsc.load_gather(tail_v, [rows, colv])
                rowr[pl.ds(srow + k * 16, 16)] = g
            pltpu.async_copy(
                rowr.at[pl.ds(srow, DIM)],
                out_hbm.at[pl.ds(p_j * DIM, DIM)],
                osem,
            )
            return m_ & jnp.logical_not(oh), no_ + 1

        _, no = lax.while_loop(cond, body, (m, no))
        return no

    def match_phase(kind, srcbase, lo_local, hi_local, n_out, pred=None):
        def vloop(v, no):
            lws, pvs, ms = [], [], []
            for t in range(4):
                voff = pl.multiple_of(v * 64 + t * 16, 8)
                packed = list_v[pl.ds(voff, 16)]
                lwv = packed & 0x7FFF
                pv = packed >> 15
                m = (lwv >= lo_local) & (lwv < hi_local)
                if pred is not None:
                    m = m & pred
                lws.append(lwv)
                pvs.append(pv)
                ms.append(m)
            m_any = (ms[0] | ms[1]) | (ms[2] | ms[3])

            def hit():
                no2 = no
                for t in range(4):
                    no2 = extract_rows(
                        kind, srcbase, lo_local, lws[t], pvs[t], ms[t], no2
                    )
                return no2

            return lax.cond(
                jnp.max(plsc.all_reduce_population_count(m_any)) > 0,
                hit,
                lambda: no,
            )

        return lax.fori_loop(0, nv4, vloop, n_out)

    def chunk_body(c, n_out):
        slot = pl.multiple_of((c % NBUF) * DIM, DIM)
        pltpu.make_async_copy(
            tt_hbm.at[:, pl.ds(0, CHUNK)], blk_v.at[pl.ds(slot, DIM)], sem
        ).wait()
        n_out = match_phase(0, slot, c * CHUNK, c * CHUNK + CHUNK, n_out)

        @pl.when(c + NBUF < NCHUNK)
        def _():
            fire(c + NBUF)

        return n_out

    n_out = lax.fori_loop(0, NCHUNK, chunk_body, jnp.int32(0))

    @pl.when(has_extra)
    def _():
        pltpu.sync_copy(
            tt_hbm.at[:, pl.ds(pl.multiple_of(col_lo + span, LANE), LANE)],
            bx_v,
        )

    n_out = match_phase(1, 0, span, span + LANE, n_out, pred=has_extra)

    @pl.when(is_tail_worker)
    def _():
        pltpu.sync_copy(tt_hbm.at[:, pl.ds(TAIL_LO, TAIL_W)], tail_v)

    n_out = match_phase(2, 0, span, span + TAIL_W, n_out, pred=is_tail_worker)

    def drain_body(r):
        pltpu.make_async_copy(
            rowr.at[pl.ds(0, DIM)], out_hbm.at[pl.ds(0, DIM)], osem
        ).wait()
        return r - 1

    lax.while_loop(lambda r: r > 0, drain_body, jnp.minimum(n_out, NROW))


def kernel(target_word, target_embedding):
    flat = _sc_gather(target_word.astype(jnp.int32), target_embedding.T)
    return flat.reshape(BATCH, DIM)


# speedup vs baseline: 1.1527x; 1.0495x over previous
import functools

import jax
import jax.numpy as jnp
from jax import lax
from jax.experimental import pallas as pl
from jax.experimental.pallas import tpu as pltpu
from jax.experimental.pallas import tpu_sc as plsc

VOCAB = 1_000_000
DIM = 64
BATCH = 16384
NUM_CORES = 2
NUM_SUBCORES = 16
NUM_WORKERS = NUM_CORES * NUM_SUBCORES
LANE = 128
NTC_FULL = VOCAB // LANE
TAIL_LO = NTC_FULL * LANE
TAIL_W = VOCAB - TAIL_LO
NTC_BASE = NTC_FULL // NUM_WORKERS
NTC_EXTRA = NTC_FULL % NUM_WORKERS
CHUNK = 512
NCHUNK = NTC_BASE * LANE // CHUNK
NBUF = 2
NROW = 8
STAGE = 4096
SENTINEL = 0x7FFF


@functools.partial(
    pl.kernel,
    mesh=plsc.VectorSubcoreMesh(core_axis_name="c", subcore_axis_name="s"),
    out_type=jax.ShapeDtypeStruct((BATCH * 2 * DIM,), jnp.float32),
    scratch_types=[
        pltpu.VMEM((STAGE,), jnp.int32),
        pltpu.VMEM((BATCH + 64,), jnp.int32),
        pltpu.VMEM((NBUF * DIM, CHUNK), jnp.float32),
        pltpu.VMEM((DIM, LANE), jnp.float32),
        pltpu.VMEM((DIM, TAIL_W), jnp.float32),
        pltpu.VMEM((NROW * DIM,), jnp.float32),
        pltpu.SemaphoreType.DMA,
        pltpu.SemaphoreType.DMA,
    ],
    compiler_params=pltpu.CompilerParams(
        disable_bounds_checks=True, needs_layout_passes=False
    ),
)
def _sc_gather(idx_hbm, tt_hbm, out_hbm, stage_v, list_v, blk_v, bx_v,
               tail_v, rowr, sem, osem):
    wid = lax.axis_index("s") * NUM_CORES + lax.axis_index("c")
    lane16 = lax.iota(jnp.int32, 16)
    has_extra = wid < NTC_EXTRA
    is_tail_worker = wid == NUM_WORKERS - 1

    tc_lo = wid * NTC_BASE + jnp.minimum(wid, NTC_EXTRA)
    col_lo = pl.multiple_of(tc_lo * LANE, LANE)
    span = NTC_BASE * LANE
    match_hi = (col_lo + span
                + jnp.where(has_extra, LANE, 0)
                + jnp.where(is_tail_worker, TAIL_W, 0))

    def fire(c):
        slot = pl.multiple_of((c % NBUF) * DIM, DIM)
        cb = pl.multiple_of(col_lo + c * CHUNK, LANE)
        pltpu.async_copy(
            tt_hbm.at[:, pl.ds(cb, CHUNK)], blk_v.at[pl.ds(slot, DIM)], sem
        )

    for b in range(NBUF):
        fire(jnp.int32(b))

    def comp_stage(s, ptr):
        soff = pl.multiple_of(s * STAGE, 8)
        pltpu.sync_copy(idx_hbm.at[pl.ds(soff, STAGE)], stage_v)

        def comp_body(i, p):
            packs, masks, cnts = [], [], []
            for t in range(4):
                off = pl.multiple_of(i * 64 + t * 16, 8)
                wv = stage_v[pl.ds(off, 16)]
                m = (wv >= col_lo) & (wv < match_hi)
                pos = s * STAGE + i * 64 + t * 16 + lane16
                packs.append((wv - col_lo) | (pos << 15))
                masks.append(m)
                cnts.append(jnp.max(plsc.all_reduce_population_count(m)))
            for t in range(4):
                plsc.store_compressed(
                    list_v.at[pl.ds(p, 16)], packs[t], mask=masks[t]
                )
                p = p + cnts[t]
            return p

        return lax.fori_loop(0, STAGE // 64, comp_body, ptr)

    cnt = lax.fori_loop(0, BATCH // STAGE, comp_stage, jnp.int32(0))
    for t in range(4):
        list_v[pl.ds(cnt + t * 16, 16)] = jnp.zeros((16,), jnp.int32) + SENTINEL
    nv4 = (cnt + 63) // 64

    def extract_rows(kind, srcbase, lo_local, lwv, pv, m, no):
        def cond(st):
            m_, _ = st
            return jnp.max(plsc.all_reduce_population_count(m_)) > 0

        def body(st):
            m_, no_ = st
            j = jnp.max(plsc.all_reduce_ffs(m_))
            oh = lane16 == j
            lw_j = jnp.max(jnp.where(oh, lwv, 0))
            p_j = jnp.max(jnp.where(oh, pv, 0))
            colv = jnp.zeros((16,), jnp.int32) + (lw_j - lo_local)

            @pl.when(no_ >= NROW)
            def _():
                pltpu.make_async_copy(
                    rowr.at[pl.ds(0, DIM)], out_hbm.at[pl.ds(0, DIM)], osem
                ).wait()

            srow = pl.multiple_of((no_ % NROW) * DIM, DIM)
            for k in range(DIM // 16):
                rows = k * 16 + lane16
                if kind == 0:
                    g = plsc.load_gather(blk_v, [srcbase + rows, colv])
                elif kind == 1:
                    g = plsc.load_gather(bx_v, [rows, colv])
                else:
                    g = plsc.load_gather(tail_v, [rows, colv])
                rowr[pl.ds(srow + k * 16, 16)] = g
            pltpu.async_copy(
                rowr.at[pl.ds(srow, DIM)],
                out_hbm.at[pl.ds(p_j * 2 * DIM, DIM)],
                osem,
            )
            return m_ & jnp.logical_not(oh), no_ + 1

        _, no = lax.while_loop(cond, body, (m, no))
        return no

    def match_phase(kind, srcbase, lo_local, hi_local, n_out, pred=None):
        def vloop(v, no):
            lws, pvs, ms = [], [], []
            for t in range(4):
                voff = pl.multiple_of(v * 64 + t * 16, 8)
                packed = list_v[pl.ds(voff, 16)]
                lwv = packed & 0x7FFF
                pv = packed >> 15
                m = (lwv >= lo_local) & (lwv < hi_local)
                if pred is not None:
                    m = m & pred
                lws.append(lwv)
                pvs.append(pv)
                ms.append(m)
            m_any = (ms[0] | ms[1]) | (ms[2] | ms[3])

            def hit():
                no2 = no
                for t in range(4):
                    no2 = extract_rows(
                        kind, srcbase, lo_local, lws[t], pvs[t], ms[t], no2
                    )
                return no2

            return lax.cond(
                jnp.max(plsc.all_reduce_population_count(m_any)) > 0,
                hit,
                lambda: no,
            )

        return lax.fori_loop(0, nv4, vloop, n_out)

    def chunk_body(c, n_out):
        slot = pl.multiple_of((c % NBUF) * DIM, DIM)
        pltpu.make_async_copy(
            tt_hbm.at[:, pl.ds(0, CHUNK)], blk_v.at[pl.ds(slot, DIM)], sem
        ).wait()
        n_out = match_phase(0, slot, c * CHUNK, c * CHUNK + CHUNK, n_out)

        @pl.when(c + NBUF < NCHUNK)
        def _():
            fire(c + NBUF)

        return n_out

    n_out = lax.fori_loop(0, NCHUNK, chunk_body, jnp.int32(0))

    @pl.when(has_extra)
    def _():
        pltpu.sync_copy(
            tt_hbm.at[:, pl.ds(pl.multiple_of(col_lo + span, LANE), LANE)],
            bx_v,
        )

    n_out = match_phase(1, 0, span, span + LANE, n_out, pred=has_extra)

    @pl.when(is_tail_worker)
    def _():
        pltpu.sync_copy(tt_hbm.at[:, pl.ds(TAIL_LO, TAIL_W)], tail_v)

    n_out = match_phase(2, 0, span, span + TAIL_W, n_out, pred=is_tail_worker)

    def drain_body(r):
        pltpu.make_async_copy(
            rowr.at[pl.ds(0, DIM)], out_hbm.at[pl.ds(0, DIM)], osem
        ).wait()
        return r - 1

    lax.while_loop(lambda r: r > 0, drain_body, jnp.minimum(n_out, NROW))


def kernel(target_word, target_embedding):
    flat = _sc_gather(target_word.astype(jnp.int32), target_embedding.T)
    return flat.reshape(BATCH, 2 * DIM)[:, :DIM]
